# Initial kernel scaffold; baseline (speedup 1.0000x reference)
import jax
import jax.numpy as jnp
from jax.experimental import pallas as pl


def kernel(x, edge_index, edge_weight, W, b):
    raise NotImplementedError("write your pallas kernel here")


# speedup vs baseline: 4.0935x; 4.0935x over previous
import functools

import jax
import jax.numpy as jnp
from jax import lax
from jax.experimental import pallas as pl
from jax.experimental.pallas import tpu as pltpu
from jax.experimental.pallas import tpu_sc as plsc

N_NODES = 10000
N_EDGES = 320000
F = 128

NC = 2
NS = 16
NL = 16
NW = NC * NS
EPW = N_EDGES // NW
ECH = 80
NCHUNK = EPW // ECH
N_PAD = 10240
RPT = N_PAD // NS
ZR = 128


def _mm_body(x_ref, w_ref, b_ref, o_ref):
    o_ref[...] = (
        jnp.dot(x_ref[...], w_ref[...], preferred_element_type=jnp.float32)
        + b_ref[...]
    )


def _matmul(x, W, b):
    bm = 1000
    return pl.pallas_call(
        _mm_body,
        grid=(N_NODES // bm,),
        in_specs=[
            pl.BlockSpec((bm, F), lambda i: (i, 0)),
            pl.BlockSpec((F, F), lambda i: (0, 0)),
            pl.BlockSpec((1, F), lambda i: (0, 0)),
        ],
        out_specs=pl.BlockSpec((bm, F), lambda i: (i, 0)),
        out_shape=jax.ShapeDtypeStruct((N_NODES, F), jnp.float32),
    )(x, W, b.reshape(1, F))


_MESH = plsc.VectorSubcoreMesh(
    core_axis_name="c", subcore_axis_name="s", num_cores=NC, num_subcores=NS
)


def _lane_bcast(v16, lane):
    return lax.gather(
        v16,
        jnp.full((NL, 1), lane, jnp.int32),
        lax.GatherDimensionNumbers(
            offset_dims=(), collapsed_slice_dims=(0,), start_index_map=(0,)
        ),
        slice_sizes=(1,),
        mode=lax.GatherScatterMode.PROMISE_IN_BOUNDS,
    )


def _spmm_body(h_hbm, src_hbm, dst_hbm, w_hbm, out_hbm,
               acc, src_v, dst_v, w_v, rows_v, zbuf, sem):
    c = lax.axis_index("c")
    s = lax.axis_index("s")
    wid = s * NC + c

    zeros16 = jnp.zeros((NL,), jnp.float32)

    def zrow(i, carry):
        for j in range(F // NL):
            zbuf[i, pl.ds(j * NL, NL)] = zeros16
        return carry

    lax.fori_loop(0, ZR, zrow, 0)

    def zacc(i, carry):
        pltpu.sync_copy(zbuf, acc.at[pl.ds(s * RPT + i * ZR, ZR)])
        return carry

    lax.fori_loop(0, RPT // ZR, zacc, 0)
    plsc.subcore_barrier()

    base = wid * EPW

    def chunk(i, carry):
        off = base + i * ECH
        pltpu.sync_copy(src_hbm.at[pl.ds(off, ECH)], src_v)
        pltpu.sync_copy(dst_hbm.at[pl.ds(off, ECH)], dst_v)
        pltpu.sync_copy(w_hbm.at[pl.ds(off, ECH)], w_v)
        pltpu.async_copy(h_hbm.at[src_v], rows_v, sem).wait()
        for r in range(ECH):
            if r % NL == 0:
                w16 = w_v[pl.ds(r, NL)]
            wb = _lane_bcast(w16, r % NL)
            for j in range(F // NL):
                sl = pl.ds(j * NL, NL)
                rows_v[r, sl] = rows_v[r, sl] * wb
        pltpu.sync_copy(rows_v, acc.at[dst_v], add=True)
        return carry

    lax.fori_loop(0, NCHUNK, chunk, 0)
    plsc.subcore_barrier()

    pltpu.sync_copy(
        acc.at[pl.ds(s * RPT, RPT)],
        out_hbm.at[c].at[pl.ds(s * RPT, RPT)],
    )


_spmm = functools.partial(
    pl.kernel,
    out_type=jax.ShapeDtypeStruct((NC, N_PAD, F), jnp.float32),
    mesh=_MESH,
    scratch_types=[
        pltpu.VMEM_SHARED((N_PAD, F), jnp.float32),
        pltpu.VMEM((ECH,), jnp.int32),
        pltpu.VMEM((ECH,), jnp.int32),
        pltpu.VMEM((ECH,), jnp.float32),
        pltpu.VMEM((ECH, F), jnp.float32),
        pltpu.VMEM((ZR, F), jnp.float32),
        pltpu.SemaphoreType.DMA,
    ],
)(_spmm_body)


def _add_body(p_ref, o_ref):
    o_ref[...] = p_ref[0] + p_ref[1]


def _pair_add(p):
    bm = 1024
    return pl.pallas_call(
        _add_body,
        grid=(N_PAD // bm,),
        in_specs=[pl.BlockSpec((NC, bm, F), lambda i: (0, i, 0))],
        out_specs=pl.BlockSpec((bm, F), lambda i: (i, 0)),
        out_shape=jax.ShapeDtypeStruct((N_PAD, F), jnp.float32),
    )(p)


def kernel(x, edge_index, edge_weight, W, b):
    h = _matmul(x, W, b)
    dst = edge_index[0].astype(jnp.int32)
    src = edge_index[1].astype(jnp.int32)
    partial = _spmm(h, src, dst, edge_weight)
    return _pair_add(partial)[:N_NODES]
